# baseline (speedup 1.0000x reference)
import functools

import jax
import jax.numpy as jnp
from jax import lax
from jax.experimental import pallas as pl
from jax.experimental.pallas import tpu as pltpu
from jax.experimental.pallas import tpu_sc as plsc

_TIME_INTERVAL = 10000.0
_N_TIME_INTERVAL = 100.0
_B = 4096
_L = 200
_EMB = 64
_NTOK = _B * _L


def _build(nw):
    tok_pw = _NTOK // nw
    chunk = 512
    nchunk = tok_pw // chunk
    nloop = nchunk // 2
    tblk = 5120
    cpb = tblk // chunk

    mesh = plsc.VectorSubcoreMesh(core_axis_name="c", subcore_axis_name="s")

    @functools.partial(
        pl.kernel,
        mesh=mesh,
        out_type=jax.ShapeDtypeStruct((_NTOK * _EMB,), jnp.float32),
        scratch_types=[
            pltpu.VMEM((tblk,), jnp.int32),
            pltpu.VMEM((tblk,), jnp.int32),
            pltpu.VMEM((101 * _EMB,), jnp.float32),
            pltpu.VMEM((chunk * _EMB,), jnp.float32),
            pltpu.VMEM((chunk * _EMB,), jnp.float32),
            pltpu.SemaphoreType.DMA,
            pltpu.SemaphoreType.DMA,
        ],
        compiler_params=pltpu.CompilerParams(use_tc_tiling_on_sc=False,
                                             needs_layout_passes=False),
    )
    def k(ts0_hbm, ts1_hbm, table_hbm, out_hbm, t0_v, t1_v, tbl_v,
          rows0, rows1, osem0, osem1):
        sid = lax.axis_index("s")
        wid = lax.axis_index("c") * 16 + sid
        tok0 = wid * tok_pw

        pltpu.sync_copy(table_hbm, tbl_v)

        lane64 = lax.iota(jnp.int32, 16) * _EMB

        def load_ts_block(blk):
            base = tok0 + blk * tblk
            pltpu.sync_copy(ts0_hbm.at[pl.ds(base, tblk)], t0_v)
            pltpu.sync_copy(ts1_hbm.at[pl.ds(base, tblk)], t1_v)

        def expand_chunk(g, rows):
            off = (g % cpb) * chunk

            def group(q, carry):
                c0 = off + q * 16
                dt = (t1_v[pl.ds(c0, 16)] - t0_v[pl.ds(c0, 16)]).astype(
                    jnp.float32)
                bix = (dt / _TIME_INTERVAL * _N_TIME_INTERVAL).astype(jnp.int32)
                sbase = jnp.maximum(bix, 0) * _EMB
                dbase = lane64 + q * (16 * _EMB)
                for c in range(_EMB):
                    v = plsc.load_gather(tbl_v, [sbase + c])
                    plsc.store_scatter(rows, [dbase + c], v)
                return carry

            lax.fori_loop(0, chunk // 16, group, 0)

        def write_chunk(g, rows, sem):
            pltpu.async_copy(
                rows, out_hbm.at[pl.ds((tok0 + g * chunk) * _EMB, chunk * _EMB)],
                sem)

        def drain_write(rows, sem):
            pltpu.make_async_copy(
                rows, out_hbm.at[pl.ds(0, chunk * _EMB)], sem).wait()

        load_ts_block(0)
        expand_chunk(0, rows0)
        write_chunk(0, rows0, osem0)

        def loop_body(gg, carry):
            g1 = gg * 2 + 1

            @pl.when((g1 % cpb) == 0)
            def _():
                load_ts_block(g1 // cpb)

            @pl.when(gg > 0)
            def _():
                drain_write(rows1, osem1)
            expand_chunk(g1, rows1)
            write_chunk(g1, rows1, osem1)

            @pl.when(gg + 1 < nloop)
            def _():
                g2 = g1 + 1

                @pl.when((g2 % cpb) == 0)
                def _():
                    load_ts_block(g2 // cpb)
                drain_write(rows0, osem0)
                expand_chunk(g2, rows0)
                write_chunk(g2, rows0, osem0)
            return carry

        lax.fori_loop(0, nloop, loop_body, 0)
        drain_write(rows0, osem0)
        drain_write(rows1, osem1)

    return k


def kernel(inputs, timestamp, W, b):
    info = plsc.get_sparse_core_info()
    nw = info.num_cores * info.num_subcores
    table = (W.T + b[None, :]).astype(jnp.float32).reshape(-1)
    ts = timestamp.astype(jnp.int32)
    ts0 = ts[:, :-1].reshape(-1)
    ts1 = ts[:, 1:].reshape(-1)
    out = _build(nw)(ts0, ts1, table)
    return out.reshape(_B, _L, _EMB)


# speedup vs baseline: 3.3616x; 3.3616x over previous
import functools

import jax
import jax.numpy as jnp
from jax import lax
from jax.experimental import pallas as pl
from jax.experimental.pallas import tpu as pltpu
from jax.experimental.pallas import tpu_sc as plsc

_TIME_INTERVAL = 10000.0
_N_TIME_INTERVAL = 100.0
_B = 4096
_L = 200
_EMB = 64
_NTOK = _B * _L


def _build(nw):
    tok_pw = _NTOK // nw
    pair_pw = tok_pw // 2
    chunk = 128
    cpair = chunk // 2
    nchunk = tok_pw // chunk
    nb = 4
    nround = nchunk // nb
    tblk = 2048
    cpb = tblk // chunk

    mesh = plsc.VectorSubcoreMesh(core_axis_name="c", subcore_axis_name="s")

    @functools.partial(
        pl.kernel,
        mesh=mesh,
        out_type=jax.ShapeDtypeStruct((_NTOK // 2, 2 * _EMB), jnp.float32),
        scratch_types=[
            pltpu.VMEM((tblk,), jnp.int32),
            pltpu.VMEM((tblk,), jnp.int32),
            pltpu.VMEM((chunk,), jnp.int32),
        ] + [pltpu.VMEM((cpair,), jnp.int32) for _ in range(nb)]
          + [pltpu.VMEM((cpair, 2 * _EMB), jnp.float32) for _ in range(nb)]
          + [pltpu.VMEM_SHARED((101 * 101, 2 * _EMB), jnp.float32)]
          + [pltpu.SemaphoreType.DMA for _ in range(2 * nb)],
        compiler_params=pltpu.CompilerParams(use_tc_tiling_on_sc=False,
                                             needs_layout_passes=False),
    )
    def k(ts0_hbm, ts1_hbm, table_hbm, out_hbm, t0_v, t1_v, tix_v,
          p0, p1, p2, p3, r0, r1, r2, r3, table_sh,
          g0, g1, g2, g3, o0, o1, o2, o3):
        pidx = [p0, p1, p2, p3]
        rows = [r0, r1, r2, r3]
        gsem = [g0, g1, g2, g3]
        osem = [o0, o1, o2, o3]

        sid = lax.axis_index("s")
        wid = lax.axis_index("c") * 16 + sid
        tok0 = wid * tok_pw
        pr0 = wid * pair_pw

        @pl.when(sid == 0)
        def _():
            pltpu.sync_copy(table_hbm, table_sh)

        iota2 = lax.iota(jnp.int32, 16) * 2

        def load_ts_block(blk):
            base = tok0 + blk * tblk
            pltpu.sync_copy(ts0_hbm.at[pl.ds(base, tblk)], t0_v)
            pltpu.sync_copy(ts1_hbm.at[pl.ds(base, tblk)], t1_v)

        def compute_pidx(g, pb):
            off = (g % cpb) * chunk
            for i in range(chunk // 16):
                c = off + i * 16
                dt = (t1_v[pl.ds(c, 16)] - t0_v[pl.ds(c, 16)]).astype(jnp.float32)
                bix = (dt / _TIME_INTERVAL * _N_TIME_INTERVAL).astype(jnp.int32)
                tix_v[pl.ds(i * 16, 16)] = jnp.maximum(bix, 0)
            for q in range(cpair // 16):
                e = plsc.load_gather(tix_v, [iota2 + q * 32])
                o = plsc.load_gather(tix_v, [iota2 + (q * 32 + 1)])
                pb[pl.ds(q * 16, 16)] = e * 101 + o

        def gather(sl):
            pltpu.async_copy(table_sh.at[pidx[sl]], rows[sl], gsem[sl])

        def drain_gather(sl):
            pltpu.make_async_copy(out_hbm.at[pl.ds(0, cpair)], rows[sl],
                                  gsem[sl]).wait()

        def write_chunk(g, sl):
            pltpu.async_copy(rows[sl], out_hbm.at[pl.ds(pr0 + g * cpair, cpair)],
                             osem[sl])

        def drain_write(sl):
            pltpu.make_async_copy(rows[sl], out_hbm.at[pl.ds(0, cpair)],
                                  osem[sl]).wait()

        plsc.subcore_barrier()

        load_ts_block(0)
        compute_pidx(0, pidx[0])
        gather(0)
        compute_pidx(1, pidx[1])
        gather(1)

        def loop_body(r, carry):
            c0 = r * nb
            for j in range(nb):
                c = c0 + j
                sl = j
                sl2 = (j + 2) % nb
                c2 = c + 2

                def issue_next(c2=c2, sl2=sl2, j=j):
                    @pl.when((c2 % cpb) == 0)
                    def _():
                        load_ts_block(c2 // cpb)
                    if j >= 2:
                        drain_write(sl2)
                    else:
                        @pl.when(r > 0)
                        def _():
                            drain_write(sl2)
                    compute_pidx(c2, pidx[sl2])
                    gather(sl2)

                if j < 2:
                    issue_next()
                else:
                    @pl.when(r + 1 < nround)
                    def _():
                        issue_next()

                drain_gather(sl)
                write_chunk(c, sl)
            return carry

        lax.fori_loop(0, nround, loop_body, 0)
        drain_write(2)
        drain_write(3)

    return k


def kernel(inputs, timestamp, W, b):
    info = plsc.get_sparse_core_info()
    nw = info.num_cores * info.num_subcores
    table = (W.T + b[None, :]).astype(jnp.float32)
    pair_table = jnp.concatenate(
        [jnp.repeat(table, 101, axis=0), jnp.tile(table, (101, 1))], axis=1)
    ts = timestamp.astype(jnp.int32)
    ts0 = ts[:, :-1].reshape(-1)
    ts1 = ts[:, 1:].reshape(-1)
    out = _build(nw)(ts0, ts1, pair_table)
    return out.reshape(_B, _L, _EMB)
